# Initial kernel scaffold; baseline (speedup 1.0000x reference)
import jax
import jax.numpy as jnp
from jax.experimental import pallas as pl


def kernel(x1, edge_index1, e1, u1, batch1, x2, edge_index2, e2, u2, batch2, params):
    raise NotImplementedError("write your pallas kernel here")


# speedup vs baseline: 6.2914x; 6.2914x over previous
import functools

import jax
import jax.numpy as jnp
from jax import lax
from jax.experimental import pallas as pl
from jax.experimental.pallas import tpu as pltpu
from jax.experimental.pallas import tpu_sc as plsc

N = 10000
E = 320000
NG = 128
FE = 16
FX = 128
FU = 32
H = 32

NC, NS = 2, 16
NW = NC * NS
CH = 128
NCHUNK = 80
EW = CH * NCHUNK
E_PAD = EW * NW
N_PAD = 10240
DUMP = 10016
NSPAN = N_PAD // NS

_mesh = plsc.VectorSubcoreMesh(core_axis_name="c", subcore_axis_name="s",
                               num_cores=NC, num_subcores=NS)
_sc_params = pltpu.CompilerParams(use_tc_tiling_on_sc=False)
_f32 = jnp.float32
_HIGHEST = lax.Precision.HIGHEST


def _dot(a, b):
    return jax.lax.dot_general(a, b, (((a.ndim - 1,), (0,)), ((), ())),
                               precision=_HIGHEST, preferred_element_type=_f32)


def _dott(a, b):
    return jax.lax.dot_general(a, b, (((0,), (0,)), ((), ())),
                               precision=_HIGHEST, preferred_element_type=_f32)


@functools.partial(
    pl.kernel,
    out_type=(jax.ShapeDtypeStruct((E_PAD, H), _f32),
              jax.ShapeDtypeStruct((E_PAD, H), _f32)),
    mesh=_mesh,
    scratch_types=(
        pltpu.VMEM((NCHUNK, CH), jnp.int32),
        pltpu.VMEM((NCHUNK, CH), jnp.int32),
        pltpu.VMEM((CH, H), _f32),
        pltpu.VMEM((CH, H), _f32),
        pltpu.SemaphoreType.DMA,
        pltpu.SemaphoreType.DMA,
    ),
    compiler_params=_sc_params,
)
def _sc_gather(gd_hbm, gs_hbm, dst_hbm, src_hbm, outd_hbm, outs_hbm,
               idx_d, idx_s, buf_d, buf_s, sem_d, sem_s):
    wid = lax.axis_index("s") * NC + lax.axis_index("c")
    base = wid * EW
    pltpu.sync_copy(dst_hbm.at[pl.ds(wid * NCHUNK, NCHUNK)], idx_d)
    pltpu.sync_copy(src_hbm.at[pl.ds(wid * NCHUNK, NCHUNK)], idx_s)

    def chunk(j, _):
        d_dma = pltpu.async_copy(gd_hbm.at[idx_d.at[j]], buf_d, sem_d)
        s_dma = pltpu.async_copy(gs_hbm.at[idx_s.at[j]], buf_s, sem_s)
        d_dma.wait()
        s_dma.wait()
        pltpu.sync_copy(buf_d, outd_hbm.at[pl.ds(base + j * CH, CH)])
        pltpu.sync_copy(buf_s, outs_hbm.at[pl.ds(base + j * CH, CH)])
        return _

    lax.fori_loop(0, NCHUNK, chunk, None)


@functools.partial(
    pl.kernel,
    out_type=jax.ShapeDtypeStruct((NC, N_PAD, FE), _f32),
    mesh=_mesh,
    scratch_types=(
        pltpu.VMEM((NCHUNK, CH), jnp.int32),
        pltpu.VMEM((CH, FE), _f32),
        pltpu.VMEM((NSPAN, FE), _f32),
        pltpu.VMEM_SHARED((N_PAD, FE), _f32),
    ),
    compiler_params=_sc_params,
)
def _sc_scatter_add(vals_hbm, dst_hbm, out_hbm, idx_v, vbuf, span, shared):
    cid = lax.axis_index("c")
    sid = lax.axis_index("s")
    wid = sid * NC + cid

    def zrow(i, _):
        span[i] = jnp.zeros((FE,), _f32)
        return _

    lax.fori_loop(0, NSPAN, zrow, None)
    pltpu.sync_copy(span, shared.at[pl.ds(sid * NSPAN, NSPAN)])
    plsc.subcore_barrier()

    pltpu.sync_copy(dst_hbm.at[pl.ds(wid * NCHUNK, NCHUNK)], idx_v)

    def chunk(j, _):
        pltpu.sync_copy(vals_hbm.at[pl.ds(wid * EW + j * CH, CH)], vbuf)
        pltpu.sync_copy(vbuf, shared.at[idx_v.at[j]], add=True)
        return _

    lax.fori_loop(0, NCHUNK, chunk, None)
    plsc.subcore_barrier()
    pltpu.sync_copy(shared.at[pl.ds(sid * NSPAN, NSPAN)], span)
    pltpu.sync_copy(span, out_hbm.at[cid, pl.ds(sid * NSPAN, NSPAN)])


@functools.partial(
    pl.kernel,
    out_type=jax.ShapeDtypeStruct((NC, N_PAD, FE), _f32),
    mesh=_mesh,
    scratch_types=(
        pltpu.VMEM((NCHUNK, CH), jnp.int32),
        pltpu.VMEM((CH, FE), _f32),
        pltpu.VMEM((NSPAN, FE), _f32),
        pltpu.VMEM_SHARED((N_PAD, FE), _f32),
    ),
    compiler_params=_sc_params,
)
def _sc_count(dst_hbm, out_hbm, idx_v, obuf, span, shared):
    cid = lax.axis_index("c")
    sid = lax.axis_index("s")
    wid = sid * NC + cid

    def zrow(i, _):
        span[i] = jnp.zeros((FE,), _f32)
        return _

    lax.fori_loop(0, NSPAN, zrow, None)

    def orow(i, _):
        obuf[i] = jnp.ones((FE,), _f32)
        return _

    lax.fori_loop(0, CH, orow, None)
    pltpu.sync_copy(span, shared.at[pl.ds(sid * NSPAN, NSPAN)])
    plsc.subcore_barrier()

    pltpu.sync_copy(dst_hbm.at[pl.ds(wid * NCHUNK, NCHUNK)], idx_v)

    def chunk(j, _):
        pltpu.sync_copy(obuf, shared.at[idx_v.at[j]], add=True)
        return _

    lax.fori_loop(0, NCHUNK, chunk, None)
    plsc.subcore_barrier()
    pltpu.sync_copy(shared.at[pl.ds(sid * NSPAN, NSPAN)], span)
    pltpu.sync_copy(span, out_hbm.at[cid, pl.ds(sid * NSPAN, NSPAN)])


NB = 1000
NGRID = N // NB
EP8 = E_PAD // 8
EB = 1280
EGRID = EP8 // EB


def _full(shape):
    return pl.BlockSpec(shape, lambda i: tuple(0 for _ in shape))


def _tc_edge_tables_body(x, b2d, uow, uot, a1, a2, au, gd, gs):
    oh = (b2d[...] == lax.broadcasted_iota(jnp.int32, (NB, NG), 1)).astype(_f32)
    tbl2 = _dot(uot[...], a2[...])
    tblc = _dot(uow[...], au[...])
    g = _dot(x[...], a1[...]) + _dot(oh, tbl2)
    gd[...] = g
    gs[...] = _dot(oh, tblc) - g


_tc_edge_tables = pl.pallas_call(
    _tc_edge_tables_body,
    grid=(NGRID,),
    in_specs=[
        pl.BlockSpec((NB, FX), lambda i: (i, 0)),
        pl.BlockSpec((NB, 1), lambda i: (i, 0)),
        _full((NG, FU)), _full((NG, FU)),
        _full((FX, H)), _full((FU, H)), _full((FU, H)),
    ],
    out_specs=[pl.BlockSpec((NB, H), lambda i: (i, 0)),
               pl.BlockSpec((NB, H), lambda i: (i, 0))],
    out_shape=[jax.ShapeDtypeStruct((N, H), _f32),
               jax.ShapeDtypeStruct((N, H), _f32)],
)


def _tc_edge_mlp_body(rd, rs, ep, ae_bd, b1t, w2_bd, b2t, out):
    pre = rd[...] + rs[...] + _dot(ep[...], ae_bd[...]) + b1t[...]
    out[...] = _dot(jnp.maximum(pre, 0.0), w2_bd[...]) + b2t[...]


_tc_edge_mlp = pl.pallas_call(
    _tc_edge_mlp_body,
    grid=(EGRID,),
    in_specs=[
        pl.BlockSpec((EB, 8 * H), lambda i: (i, 0)),
        pl.BlockSpec((EB, 8 * H), lambda i: (i, 0)),
        pl.BlockSpec((EB, 8 * FE), lambda i: (i, 0)),
        _full((8 * FE, 8 * H)), _full((1, 8 * H)),
        _full((8 * H, 8 * FE)), _full((1, 8 * FE)),
    ],
    out_specs=pl.BlockSpec((EB, 8 * FE), lambda i: (i, 0)),
    out_shape=jax.ShapeDtypeStruct((EP8, 8 * FE), _f32),
)


def _tc_node_body(x, b2d, sp, cp, uow, uot,
                  b1w, b2w, buw, bew, b1n, w2n, b2n,
                  c1w, c2w, b1a, w2a1, w2a2, b2a1, b2a2,
                  cb, g1a, g1b, g1c, g2w, g2c,
                  xo, us1, us2, uo):
    i = pl.program_id(0)
    oh = (b2d[...] == lax.broadcasted_iota(jnp.int32, (NB, NG), 1)).astype(_f32)
    e_agg = (sp[0] + sp[1]) / jnp.clip(cp[0] + cp[1], 1.0, None)
    tbl_b = _dot(uot[...], b2w[...]) + _dot(uow[...], buw[...])
    x_pre = (_dot(x[...], b1w[...]) + _dot(oh, tbl_b)
             + _dot(e_agg, bew[...]) + b1n[...])
    x_new = _dot(jnp.maximum(x_pre, 0.0), w2n[...]) + b2n[...]
    xo[...] = x_new
    ub = _dot(oh, uow[...])
    pre_a = _dot(x_new, c1w[...]) + _dot(ub, c2w[...]) + b1a[...]
    ah = jnp.maximum(pre_a, 0.0)
    a1 = jax.nn.sigmoid(_dot(ah, w2a1[...]) + b2a1[...])
    a2 = jax.nn.sigmoid(_dot(ah, w2a2[...]) + b2a2[...])
    w1 = a1 * x_new
    w2 = a2 * ub

    @pl.when(i == 0)
    def _init():
        us1[...] = jnp.zeros_like(us1)
        us2[...] = jnp.zeros_like(us2)

    us1[...] += _dott(oh, w1)
    us2[...] += _dott(oh, w2)

    @pl.when(i == NGRID - 1)
    def _glob():
        inv = 1.0 / jnp.clip(cb[...], 1.0, None)
        ua1 = us1[...] * inv
        ua2 = us2[...] * inv[:, :FU]
        pre = _dot(ua1, g1a[...]) + _dot(ua2, g1b[...]) + g1c[...]
        uo[...] = _dot(jnp.maximum(pre, 0.0), g2w[...]) + g2c[...]


_tc_node = pl.pallas_call(
    _tc_node_body,
    grid=(NGRID,),
    in_specs=[
        pl.BlockSpec((NB, FX), lambda i: (i, 0)),
        pl.BlockSpec((NB, 1), lambda i: (i, 0)),
        pl.BlockSpec((NC, NB, FE), lambda i: (0, i, 0)),
        pl.BlockSpec((NC, NB, FE), lambda i: (0, i, 0)),
        _full((NG, FU)), _full((NG, FU)),
        _full((FX, H)), _full((FU, H)), _full((FU, H)), _full((FE, H)),
        _full((1, H)), _full((H, FX)), _full((1, FX)),
        _full((FX, H)), _full((FU, H)), _full((1, H)),
        _full((H, FX)), _full((H, FU)), _full((1, FX)), _full((1, FU)),
        _full((NG, NG)), _full((FX, H)), _full((FU, H)), _full((1, H)),
        _full((H, FU)), _full((1, FU)),
    ],
    out_specs=[pl.BlockSpec((NB, FX), lambda i: (i, 0)),
               _full((NG, FX)), _full((NG, FU)), _full((NG, FU))],
    out_shape=[jax.ShapeDtypeStruct((N, FX), _f32),
               jax.ShapeDtypeStruct((NG, FX), _f32),
               jax.ShapeDtypeStruct((NG, FU), _f32),
               jax.ShapeDtypeStruct((NG, FU), _f32)],
)


def _tc_final_body(u1, u2, f1a, f1b, f1c, f2w, f2c, out):
    pre = _dot(u1[...], f1a[...]) + _dot(u2[...], f1b[...]) + f1c[...]
    out[...] = _dot(jnp.maximum(pre, 0.0), f2w[...]) + f2c[...]


_tc_final = pl.pallas_call(
    _tc_final_body,
    grid=(1,),
    in_specs=[_full((NG, FU)), _full((NG, FU)),
              _full((FU, H)), _full((FU, H)), _full((1, H)),
              _full((H, 64)), _full((1, 64))],
    out_specs=_full((NG, 64)),
    out_shape=jax.ShapeDtypeStruct((NG, 64), _f32),
)


def _tc_cntb_body(b2d, out):
    i = pl.program_id(0)
    oh = (b2d[...] == lax.broadcasted_iota(jnp.int32, (NB, NG), 1)).astype(_f32)

    @pl.when(i == 0)
    def _init():
        out[...] = jnp.zeros_like(out)

    out[...] += _dott(oh, jnp.ones((NB, NG), _f32))


_tc_cntb = pl.pallas_call(
    _tc_cntb_body,
    grid=(NGRID,),
    in_specs=[pl.BlockSpec((NB, 1), lambda i: (i, 0))],
    out_specs=_full((NG, NG)),
    out_shape=jax.ShapeDtypeStruct((NG, NG), _f32),
)


def _block_diag8(w):
    f_in, f_out = w.shape
    return (jnp.eye(8, dtype=_f32)[:, None, :, None] * w[None, :, None, :]
            ).reshape(8 * f_in, 8 * f_out)


def _pad_idx(idx, fill):
    pad = jnp.full((E_PAD - E,), fill, jnp.int32)
    return jnp.concatenate([idx, pad]).reshape(E_PAD // CH, CH)


def kernel(x1, edge_index1, e1, u1, batch1, x2, edge_index2, e2, u2, batch2,
           params):
    (w1e, b1e), (w2e, b2e) = params['edge']
    (w1n, b1n), (w2n, b2n) = params['node']
    (w1a, b1a), (w2a, b2a) = params['attn']
    (g1w, g1c), (g2w, g2c) = params['glob']
    (f1w, f1c), (f2w, f2c) = params['final']

    ae_bd = _block_diag8(w1e[160:176])
    w2e_bd = _block_diag8(w2e)
    b1e_t = jnp.tile(b1e, 8)[None, :]
    b2e_t = jnp.tile(b2e, 8)[None, :]

    edge_w = (w1e[0:128], w1e[128:160], w1e[176:208])
    node_w = (w1n[0:128], w1n[128:160], w1n[176:208], w1n[160:176],
              b1n[None, :], w2n, b2n[None, :])
    attn_w = (w1a[0:128], w1a[128:160], b1a[None, :],
              w2a[:, :128], w2a[:, 128:], b2a[None, :128], b2a[None, 128:])
    glob_w = (g1w[:128], g1w[128:], g1c[None, :], g2w, g2c[None, :])
    final_w = (f1w[:32], f1w[32:], f1c[None, :], f2w, f2c[None, :])

    def prep_graph(edge_index, batch, e):
        dst, src = edge_index[1], edge_index[0]
        dst_g = _pad_idx(dst, 0)
        src_g = _pad_idx(src, 0)
        dst_s = _pad_idx(dst, DUMP)
        b2d = batch[:, None]
        e_p = jnp.concatenate(
            [e, jnp.zeros((E_PAD - E, FE), _f32)]).reshape(EP8, 8 * FE)
        cnt = _sc_count(dst_s)
        cb = _tc_cntb(b2d)
        return dict(dst_g=dst_g, src_g=src_g, dst_s=dst_s, b2d=b2d,
                    e_p=e_p, cnt=cnt, cb=cb)

    g1 = prep_graph(edge_index1, batch1, e1)
    g2 = prep_graph(edge_index2, batch2, e2)

    def gnn(g, x, u_own, u_other):
        gd, gs = _tc_edge_tables(x, g['b2d'], u_own, u_other, *edge_w)
        rd, rs = _sc_gather(gd, gs, g['dst_g'], g['src_g'])
        e_new_p = _tc_edge_mlp(rd.reshape(EP8, 8 * H), rs.reshape(EP8, 8 * H),
                               g['e_p'], ae_bd, b1e_t, w2e_bd, b2e_t)
        s_part = _sc_scatter_add(e_new_p.reshape(E_PAD, FE), g['dst_s'])
        x_new, _, _, u_new = _tc_node(x, g['b2d'], s_part, g['cnt'],
                                      u_own, u_other, *node_w, *attn_w,
                                      g['cb'], *glob_w)
        return x_new, e_new_p, u_new

    outs = []
    for _ in range(2):
        x1, e_new1, u1 = gnn(g1, x1, u1, u2)
        g1['e_p'] = e_new1
        x2, e_new2, u2 = gnn(g2, x2, u2, u1)
        g2['e_p'] = e_new2
        outs.append(_tc_final(u1, u2, *final_w))
    return jnp.stack(outs)


# speedup vs baseline: 7.8680x; 1.2506x over previous
import functools

import jax
import jax.numpy as jnp
from jax import lax
from jax.experimental import pallas as pl
from jax.experimental.pallas import tpu as pltpu
from jax.experimental.pallas import tpu_sc as plsc

N = 10000
E = 320000
NG = 128
FE = 16
FX = 128
FU = 32
H = 32

NC, NS = 2, 16
NW = NC * NS
CH = 128
NCHUNK = 80
EW = CH * NCHUNK
E_PAD = EW * NW
N_PAD = 10240
DUMP = 10016
NSPAN = N_PAD // NS

_mesh = plsc.VectorSubcoreMesh(core_axis_name="c", subcore_axis_name="s",
                               num_cores=NC, num_subcores=NS)
_sc_params = pltpu.CompilerParams(use_tc_tiling_on_sc=False)
_f32 = jnp.float32
_HIGHEST = lax.Precision.HIGHEST


def _dot(a, b):
    return jax.lax.dot_general(a, b, (((a.ndim - 1,), (0,)), ((), ())),
                               precision=_HIGHEST, preferred_element_type=_f32)


def _dotd(a, b):
    return jax.lax.dot_general(a, b, (((a.ndim - 1,), (0,)), ((), ())),
                               preferred_element_type=_f32)


def _dott(a, b):
    return jax.lax.dot_general(a, b, (((0,), (0,)), ((), ())),
                               precision=_HIGHEST, preferred_element_type=_f32)


@functools.partial(
    pl.kernel,
    out_type=(jax.ShapeDtypeStruct((E_PAD, H), _f32),
              jax.ShapeDtypeStruct((E_PAD, H), _f32)),
    mesh=_mesh,
    scratch_types=(
        pltpu.VMEM((NCHUNK, CH), jnp.int32),
        pltpu.VMEM((NCHUNK, CH), jnp.int32),
        pltpu.VMEM((CH, H), _f32),
        pltpu.VMEM((CH, H), _f32),
        pltpu.VMEM((CH, H), _f32),
        pltpu.VMEM((CH, H), _f32),
        pltpu.SemaphoreType.DMA,
        pltpu.SemaphoreType.DMA,
        pltpu.SemaphoreType.DMA,
        pltpu.SemaphoreType.DMA,
        pltpu.SemaphoreType.DMA,
        pltpu.SemaphoreType.DMA,
        pltpu.SemaphoreType.DMA,
        pltpu.SemaphoreType.DMA,
    ),
    compiler_params=_sc_params,
)
def _sc_gather(gd_hbm, gs_hbm, dst_hbm, src_hbm, outd_hbm, outs_hbm,
               idx_d, idx_s, bd0, bs0, bd1, bs1,
               sgd0, sgs0, sgd1, sgs1, sod0, sos0, sod1, sos1):
    wid = lax.axis_index("s") * NC + lax.axis_index("c")
    base = wid * EW
    pltpu.sync_copy(dst_hbm.at[pl.ds(wid * NCHUNK, NCHUNK)], idx_d)
    pltpu.sync_copy(src_hbm.at[pl.ds(wid * NCHUNK, NCHUNK)], idx_s)

    sets = ((bd0, bs0, sgd0, sgs0, sod0, sos0),
            (bd1, bs1, sgd1, sgs1, sod1, sos1))

    def g(j, t):
        bd, bs, sgd, sgs, _, _ = sets[t]
        pltpu.async_copy(gd_hbm.at[idx_d.at[j]], bd, sgd)
        pltpu.async_copy(gs_hbm.at[idx_s.at[j]], bs, sgs)

    def wg(t):
        bd, bs, sgd, sgs, _, _ = sets[t]
        pltpu.make_async_copy(gd_hbm.at[pl.ds(0, CH)], bd, sgd).wait()
        pltpu.make_async_copy(gs_hbm.at[pl.ds(0, CH)], bs, sgs).wait()

    def o(j, t):
        bd, bs, _, _, sod, sos = sets[t]
        pltpu.async_copy(bd, outd_hbm.at[pl.ds(base + j * CH, CH)], sod)
        pltpu.async_copy(bs, outs_hbm.at[pl.ds(base + j * CH, CH)], sos)

    def wo(t):
        bd, bs, _, _, sod, sos = sets[t]
        pltpu.make_async_copy(bd, outd_hbm.at[pl.ds(base, CH)], sod).wait()
        pltpu.make_async_copy(bs, outs_hbm.at[pl.ds(base, CH)], sos).wait()

    g(0, 0)
    g(1, 1)
    wg(0)
    o(0, 0)

    def body(k, _):
        j = 2 * k + 1
        wo(0)
        g(j + 1, 0)
        wg(1)
        o(j, 1)
        wo(1)
        g(j + 2, 1)
        wg(0)
        o(j + 1, 0)
        return _

    lax.fori_loop(0, (NCHUNK - 2) // 2, body, None)
    wg(1)
    o(NCHUNK - 1, 1)
    wo(0)
    wo(1)


@functools.partial(
    pl.kernel,
    out_type=jax.ShapeDtypeStruct((NC, N_PAD, FE), _f32),
    mesh=_mesh,
    scratch_types=(
        pltpu.VMEM((NCHUNK, CH), jnp.int32),
        pltpu.VMEM((CH, FE), _f32),
        pltpu.VMEM((CH, FE), _f32),
        pltpu.VMEM((NSPAN, FE), _f32),
        pltpu.VMEM_SHARED((N_PAD, FE), _f32),
        pltpu.SemaphoreType.DMA,
        pltpu.SemaphoreType.DMA,
    ),
    compiler_params=_sc_params,
)
def _sc_scatter_add(vals_hbm, dst_hbm, out_hbm, idx_v, vb0, vb1, span, shared,
                    sv0, sv1):
    cid = lax.axis_index("c")
    sid = lax.axis_index("s")
    wid = sid * NC + cid

    def zrow(i, _):
        span[i] = jnp.zeros((FE,), _f32)
        return _

    lax.fori_loop(0, NSPAN, zrow, None)
    pltpu.sync_copy(span, shared.at[pl.ds(sid * NSPAN, NSPAN)])
    plsc.subcore_barrier()

    pltpu.sync_copy(dst_hbm.at[pl.ds(wid * NCHUNK, NCHUNK)], idx_v)

    sets = ((vb0, sv0), (vb1, sv1))

    def v(j, t):
        vb, sv = sets[t]
        pltpu.async_copy(vals_hbm.at[pl.ds(wid * EW + j * CH, CH)], vb, sv)

    def wv(t):
        vb, sv = sets[t]
        pltpu.make_async_copy(vals_hbm.at[pl.ds(wid * EW, CH)], vb, sv).wait()

    def s(j, t):
        vb, _ = sets[t]
        pltpu.sync_copy(vb, shared.at[idx_v.at[j]], add=True)

    def chunk(j, _):
        pltpu.sync_copy(vals_hbm.at[pl.ds(wid * EW + j * CH, CH)], vb0)
        pltpu.sync_copy(vb0, shared.at[idx_v.at[j]], add=True)
        return _

    lax.fori_loop(0, NCHUNK, chunk, None)
    plsc.subcore_barrier()
    pltpu.sync_copy(shared.at[pl.ds(sid * NSPAN, NSPAN)], span)
    pltpu.sync_copy(span, out_hbm.at[cid, pl.ds(sid * NSPAN, NSPAN)])


@functools.partial(
    pl.kernel,
    out_type=jax.ShapeDtypeStruct((NC, N_PAD, FE), _f32),
    mesh=_mesh,
    scratch_types=(
        pltpu.VMEM((NCHUNK, CH), jnp.int32),
        pltpu.VMEM((CH, FE), _f32),
        pltpu.VMEM((NSPAN, FE), _f32),
        pltpu.VMEM_SHARED((N_PAD, FE), _f32),
    ),
    compiler_params=_sc_params,
)
def _sc_count(dst_hbm, out_hbm, idx_v, obuf, span, shared):
    cid = lax.axis_index("c")
    sid = lax.axis_index("s")
    wid = sid * NC + cid

    def zrow(i, _):
        span[i] = jnp.zeros((FE,), _f32)
        return _

    lax.fori_loop(0, NSPAN, zrow, None)

    def orow(i, _):
        obuf[i] = jnp.ones((FE,), _f32)
        return _

    lax.fori_loop(0, CH, orow, None)
    pltpu.sync_copy(span, shared.at[pl.ds(sid * NSPAN, NSPAN)])
    plsc.subcore_barrier()

    pltpu.sync_copy(dst_hbm.at[pl.ds(wid * NCHUNK, NCHUNK)], idx_v)

    def chunk(j, _):
        pltpu.sync_copy(obuf, shared.at[idx_v.at[j]], add=True)
        return _

    lax.fori_loop(0, NCHUNK, chunk, None)
    plsc.subcore_barrier()
    pltpu.sync_copy(shared.at[pl.ds(sid * NSPAN, NSPAN)], span)
    pltpu.sync_copy(span, out_hbm.at[cid, pl.ds(sid * NSPAN, NSPAN)])


NB = 1000
NGRID = N // NB
EP8 = E_PAD // 8
EB = 1280
EGRID = EP8 // EB


def _full(shape):
    return pl.BlockSpec(shape, lambda i: tuple(0 for _ in shape))


def _tc_edge_tables_body(x, b2d, uow, uot, a1, a2, au, gd, gs):
    oh = (b2d[...] == lax.broadcasted_iota(jnp.int32, (NB, NG), 1)).astype(_f32)
    tbl2 = _dotd(uot[...], a2[...])
    tblc = _dotd(uow[...], au[...])
    g = _dotd(x[...], a1[...]) + _dot(oh, tbl2)
    gd[...] = g
    gs[...] = _dot(oh, tblc) - g


_tc_edge_tables = pl.pallas_call(
    _tc_edge_tables_body,
    grid=(NGRID,),
    in_specs=[
        pl.BlockSpec((NB, FX), lambda i: (i, 0)),
        pl.BlockSpec((NB, 1), lambda i: (i, 0)),
        _full((NG, FU)), _full((NG, FU)),
        _full((FX, H)), _full((FU, H)), _full((FU, H)),
    ],
    out_specs=[pl.BlockSpec((NB, H), lambda i: (i, 0)),
               pl.BlockSpec((NB, H), lambda i: (i, 0))],
    out_shape=[jax.ShapeDtypeStruct((N, H), _f32),
               jax.ShapeDtypeStruct((N, H), _f32)],
)


def _tc_edge_mlp_body(rd, rs, ep, ae_bd, b1t, w2_bd, b2t, out):
    pre = rd[...] + rs[...] + _dotd(ep[...], ae_bd[...]) + b1t[...]
    out[...] = _dotd(jnp.maximum(pre, 0.0), w2_bd[...]) + b2t[...]


_tc_edge_mlp = pl.pallas_call(
    _tc_edge_mlp_body,
    grid=(EGRID,),
    in_specs=[
        pl.BlockSpec((EB, 8 * H), lambda i: (i, 0)),
        pl.BlockSpec((EB, 8 * H), lambda i: (i, 0)),
        pl.BlockSpec((EB, 8 * FE), lambda i: (i, 0)),
        _full((8 * FE, 8 * H)), _full((1, 8 * H)),
        _full((8 * H, 8 * FE)), _full((1, 8 * FE)),
    ],
    out_specs=pl.BlockSpec((EB, 8 * FE), lambda i: (i, 0)),
    out_shape=jax.ShapeDtypeStruct((EP8, 8 * FE), _f32),
)


def _tc_node_body(x, b2d, sp, cp, uow, uot,
                  b1w, b2w, buw, bew, b1n, w2n, b2n,
                  c1w, c2w, b1a, w2a1, w2a2, b2a1, b2a2,
                  cb, g1a, g1b, g1c, g2w, g2c,
                  xo, us1, us2, uo):
    i = pl.program_id(0)
    oh = (b2d[...] == lax.broadcasted_iota(jnp.int32, (NB, NG), 1)).astype(_f32)
    e_agg = (sp[0] + sp[1]) / jnp.clip(cp[0] + cp[1], 1.0, None)
    tbl_b = _dotd(uot[...], b2w[...]) + _dotd(uow[...], buw[...])
    x_pre = (_dotd(x[...], b1w[...]) + _dot(oh, tbl_b)
             + _dotd(e_agg, bew[...]) + b1n[...])
    x_new = _dotd(jnp.maximum(x_pre, 0.0), w2n[...]) + b2n[...]
    xo[...] = x_new
    ub = _dot(oh, uow[...])
    pre_a = _dotd(x_new, c1w[...]) + _dotd(ub, c2w[...]) + b1a[...]
    ah = jnp.maximum(pre_a, 0.0)
    a1 = jax.nn.sigmoid(_dotd(ah, w2a1[...]) + b2a1[...])
    a2 = jax.nn.sigmoid(_dotd(ah, w2a2[...]) + b2a2[...])
    w1 = a1 * x_new
    w2 = a2 * ub

    @pl.when(i == 0)
    def _init():
        us1[...] = jnp.zeros_like(us1)
        us2[...] = jnp.zeros_like(us2)

    us1[...] += _dott(oh, w1)
    us2[...] += _dott(oh, w2)

    @pl.when(i == NGRID - 1)
    def _glob():
        inv = 1.0 / jnp.clip(cb[...], 1.0, None)
        ua1 = us1[...] * inv
        ua2 = us2[...] * inv[:, :FU]
        pre = _dotd(ua1, g1a[...]) + _dotd(ua2, g1b[...]) + g1c[...]
        uo[...] = _dotd(jnp.maximum(pre, 0.0), g2w[...]) + g2c[...]


_tc_node = pl.pallas_call(
    _tc_node_body,
    grid=(NGRID,),
    in_specs=[
        pl.BlockSpec((NB, FX), lambda i: (i, 0)),
        pl.BlockSpec((NB, 1), lambda i: (i, 0)),
        pl.BlockSpec((NC, NB, FE), lambda i: (0, i, 0)),
        pl.BlockSpec((NC, NB, FE), lambda i: (0, i, 0)),
        _full((NG, FU)), _full((NG, FU)),
        _full((FX, H)), _full((FU, H)), _full((FU, H)), _full((FE, H)),
        _full((1, H)), _full((H, FX)), _full((1, FX)),
        _full((FX, H)), _full((FU, H)), _full((1, H)),
        _full((H, FX)), _full((H, FU)), _full((1, FX)), _full((1, FU)),
        _full((NG, NG)), _full((FX, H)), _full((FU, H)), _full((1, H)),
        _full((H, FU)), _full((1, FU)),
    ],
    out_specs=[pl.BlockSpec((NB, FX), lambda i: (i, 0)),
               _full((NG, FX)), _full((NG, FU)), _full((NG, FU))],
    out_shape=[jax.ShapeDtypeStruct((N, FX), _f32),
               jax.ShapeDtypeStruct((NG, FX), _f32),
               jax.ShapeDtypeStruct((NG, FU), _f32),
               jax.ShapeDtypeStruct((NG, FU), _f32)],
)


def _tc_final_body(u1, u2, f1a, f1b, f1c, f2w, f2c, out):
    pre = _dotd(u1[...], f1a[...]) + _dotd(u2[...], f1b[...]) + f1c[...]
    out[...] = _dotd(jnp.maximum(pre, 0.0), f2w[...]) + f2c[...]


_tc_final = pl.pallas_call(
    _tc_final_body,
    grid=(1,),
    in_specs=[_full((NG, FU)), _full((NG, FU)),
              _full((FU, H)), _full((FU, H)), _full((1, H)),
              _full((H, 64)), _full((1, 64))],
    out_specs=_full((NG, 64)),
    out_shape=jax.ShapeDtypeStruct((NG, 64), _f32),
)


def _tc_cntb_body(b2d, out):
    i = pl.program_id(0)
    oh = (b2d[...] == lax.broadcasted_iota(jnp.int32, (NB, NG), 1)).astype(_f32)

    @pl.when(i == 0)
    def _init():
        out[...] = jnp.zeros_like(out)

    out[...] += _dott(oh, jnp.ones((NB, NG), _f32))


_tc_cntb = pl.pallas_call(
    _tc_cntb_body,
    grid=(NGRID,),
    in_specs=[pl.BlockSpec((NB, 1), lambda i: (i, 0))],
    out_specs=_full((NG, NG)),
    out_shape=jax.ShapeDtypeStruct((NG, NG), _f32),
)


def _block_diag8(w):
    f_in, f_out = w.shape
    return (jnp.eye(8, dtype=_f32)[:, None, :, None] * w[None, :, None, :]
            ).reshape(8 * f_in, 8 * f_out)


def _pad_idx(idx, fill):
    pad = jnp.full((E_PAD - E,), fill, jnp.int32)
    return jnp.concatenate([idx, pad]).reshape(E_PAD // CH, CH)


def kernel(x1, edge_index1, e1, u1, batch1, x2, edge_index2, e2, u2, batch2,
           params):
    (w1e, b1e), (w2e, b2e) = params['edge']
    (w1n, b1n), (w2n, b2n) = params['node']
    (w1a, b1a), (w2a, b2a) = params['attn']
    (g1w, g1c), (g2w, g2c) = params['glob']
    (f1w, f1c), (f2w, f2c) = params['final']

    ae_bd = _block_diag8(w1e[160:176])
    w2e_bd = _block_diag8(w2e)
    b1e_t = jnp.tile(b1e, 8)[None, :]
    b2e_t = jnp.tile(b2e, 8)[None, :]

    edge_w = (w1e[0:128], w1e[128:160], w1e[176:208])
    node_w = (w1n[0:128], w1n[128:160], w1n[176:208], w1n[160:176],
              b1n[None, :], w2n, b2n[None, :])
    attn_w = (w1a[0:128], w1a[128:160], b1a[None, :],
              w2a[:, :128], w2a[:, 128:], b2a[None, :128], b2a[None, 128:])
    glob_w = (g1w[:128], g1w[128:], g1c[None, :], g2w, g2c[None, :])
    final_w = (f1w[:32], f1w[32:], f1c[None, :], f2w, f2c[None, :])

    def prep_graph(edge_index, batch, e):
        dst, src = edge_index[1], edge_index[0]
        dst_g = _pad_idx(dst, 0)
        src_g = _pad_idx(src, 0)
        dst_s = _pad_idx(dst, DUMP)
        b2d = batch[:, None]
        e_p = jnp.concatenate(
            [e, jnp.zeros((E_PAD - E, FE), _f32)]).reshape(EP8, 8 * FE)
        cnt = _sc_count(dst_s)
        cb = _tc_cntb(b2d)
        return dict(dst_g=dst_g, src_g=src_g, dst_s=dst_s, b2d=b2d,
                    e_p=e_p, cnt=cnt, cb=cb)

    g1 = prep_graph(edge_index1, batch1, e1)
    g2 = prep_graph(edge_index2, batch2, e2)

    def gnn(g, x, u_own, u_other):
        gd, gs = _tc_edge_tables(x, g['b2d'], u_own, u_other, *edge_w)
        rd, rs = _sc_gather(gd, gs, g['dst_g'], g['src_g'])
        e_new_p = _tc_edge_mlp(rd.reshape(EP8, 8 * H), rs.reshape(EP8, 8 * H),
                               g['e_p'], ae_bd, b1e_t, w2e_bd, b2e_t)
        s_part = _sc_scatter_add(e_new_p.reshape(E_PAD, FE), g['dst_s'])
        x_new, _, _, u_new = _tc_node(x, g['b2d'], s_part, g['cnt'],
                                      u_own, u_other, *node_w, *attn_w,
                                      g['cb'], *glob_w)
        return x_new, e_new_p, u_new

    outs = []
    for _ in range(2):
        x1, e_new1, u1 = gnn(g1, x1, u1, u2)
        g1['e_p'] = e_new1
        x2, e_new2, u2 = gnn(g2, x2, u2, u1)
        g2['e_p'] = e_new2
        outs.append(_tc_final(u1, u2, *final_w))
    return jnp.stack(outs)


# speedup vs baseline: 8.3063x; 1.0557x over previous
import functools

import jax
import jax.numpy as jnp
from jax import lax
from jax.experimental import pallas as pl
from jax.experimental.pallas import tpu as pltpu
from jax.experimental.pallas import tpu_sc as plsc

N = 10000
E = 320000
NG = 128
FE = 16
FX = 128
FU = 32
H = 32

NC, NS = 2, 16
NW = NC * NS
CH = 128
NCHUNK = 80
EW = CH * NCHUNK
E_PAD = EW * NW
N_PAD = 10240
DUMP = 10016
NSPAN = N_PAD // NS

_mesh = plsc.VectorSubcoreMesh(core_axis_name="c", subcore_axis_name="s",
                               num_cores=NC, num_subcores=NS)
_sc_params = pltpu.CompilerParams(use_tc_tiling_on_sc=False)
_f32 = jnp.float32
_HIGHEST = lax.Precision.HIGHEST


def _dot(a, b):
    return jax.lax.dot_general(a, b, (((a.ndim - 1,), (0,)), ((), ())),
                               precision=_HIGHEST, preferred_element_type=_f32)


def _dotd(a, b):
    return jax.lax.dot_general(a, b, (((a.ndim - 1,), (0,)), ((), ())),
                               preferred_element_type=_f32)


def _dott(a, b):
    return jax.lax.dot_general(a, b, (((0,), (0,)), ((), ())),
                               precision=_HIGHEST, preferred_element_type=_f32)


@functools.partial(
    pl.kernel,
    out_type=(jax.ShapeDtypeStruct((E_PAD, H), _f32),
              jax.ShapeDtypeStruct((E_PAD, H), _f32)),
    mesh=_mesh,
    scratch_types=(
        pltpu.VMEM((NCHUNK, CH), jnp.int32),
        pltpu.VMEM((NCHUNK, CH), jnp.int32),
        pltpu.VMEM((CH, H), _f32),
        pltpu.VMEM((CH, H), _f32),
        pltpu.VMEM((CH, H), _f32),
        pltpu.VMEM((CH, H), _f32),
        pltpu.SemaphoreType.DMA,
        pltpu.SemaphoreType.DMA,
        pltpu.SemaphoreType.DMA,
        pltpu.SemaphoreType.DMA,
        pltpu.SemaphoreType.DMA,
        pltpu.SemaphoreType.DMA,
        pltpu.SemaphoreType.DMA,
        pltpu.SemaphoreType.DMA,
    ),
    compiler_params=_sc_params,
)
def _sc_gather(gd_hbm, gs_hbm, dst_hbm, src_hbm, outd_hbm, outs_hbm,
               idx_d, idx_s, bd0, bs0, bd1, bs1,
               sgd0, sgs0, sgd1, sgs1, sod0, sos0, sod1, sos1):
    wid = lax.axis_index("s") * NC + lax.axis_index("c")
    base = wid * EW
    pltpu.sync_copy(dst_hbm.at[pl.ds(wid * NCHUNK, NCHUNK)], idx_d)
    pltpu.sync_copy(src_hbm.at[pl.ds(wid * NCHUNK, NCHUNK)], idx_s)

    sets = ((bd0, bs0, sgd0, sgs0, sod0, sos0),
            (bd1, bs1, sgd1, sgs1, sod1, sos1))

    def g(j, t):
        bd, bs, sgd, sgs, _, _ = sets[t]
        pltpu.async_copy(gd_hbm.at[idx_d.at[j]], bd, sgd)
        pltpu.async_copy(gs_hbm.at[idx_s.at[j]], bs, sgs)

    def wg(t):
        bd, bs, sgd, sgs, _, _ = sets[t]
        pltpu.make_async_copy(gd_hbm.at[pl.ds(0, CH)], bd, sgd).wait()
        pltpu.make_async_copy(gs_hbm.at[pl.ds(0, CH)], bs, sgs).wait()

    def o(j, t):
        bd, bs, _, _, sod, sos = sets[t]
        pltpu.async_copy(bd, outd_hbm.at[pl.ds(base + j * CH, CH)], sod)
        pltpu.async_copy(bs, outs_hbm.at[pl.ds(base + j * CH, CH)], sos)

    def wo(t):
        bd, bs, _, _, sod, sos = sets[t]
        pltpu.make_async_copy(bd, outd_hbm.at[pl.ds(base, CH)], sod).wait()
        pltpu.make_async_copy(bs, outs_hbm.at[pl.ds(base, CH)], sos).wait()

    g(0, 0)
    g(1, 1)
    wg(0)
    o(0, 0)

    def body(k, _):
        j = 2 * k + 1
        wo(0)
        g(j + 1, 0)
        wg(1)
        o(j, 1)
        wo(1)
        g(j + 2, 1)
        wg(0)
        o(j + 1, 0)
        return _

    lax.fori_loop(0, (NCHUNK - 2) // 2, body, None)
    wg(1)
    o(NCHUNK - 1, 1)
    wo(0)
    wo(1)


@functools.partial(
    pl.kernel,
    out_type=jax.ShapeDtypeStruct((NC, N_PAD, FE), _f32),
    mesh=_mesh,
    scratch_types=(
        pltpu.VMEM((NCHUNK, CH), jnp.int32),
        pltpu.VMEM((CH, FE), _f32),
        pltpu.VMEM((CH, FE), _f32),
        pltpu.VMEM((NSPAN, FE), _f32),
        pltpu.VMEM_SHARED((N_PAD, FE), _f32),
        pltpu.SemaphoreType.DMA,
        pltpu.SemaphoreType.DMA,
    ),
    compiler_params=_sc_params,
)
def _sc_scatter_add(vals_hbm, dst_hbm, out_hbm, idx_v, vb0, vb1, span, shared,
                    sv0, sv1):
    cid = lax.axis_index("c")
    sid = lax.axis_index("s")
    wid = sid * NC + cid

    def zrow(i, _):
        span[i] = jnp.zeros((FE,), _f32)
        return _

    lax.fori_loop(0, NSPAN, zrow, None)
    pltpu.sync_copy(span, shared.at[pl.ds(sid * NSPAN, NSPAN)])
    plsc.subcore_barrier()

    pltpu.sync_copy(dst_hbm.at[pl.ds(wid * NCHUNK, NCHUNK)], idx_v)

    sets = ((vb0, sv0), (vb1, sv1))

    def v(j, t):
        vb, sv = sets[t]
        pltpu.async_copy(vals_hbm.at[pl.ds(wid * EW + j * CH, CH)], vb, sv)

    def wv(t):
        vb, sv = sets[t]
        pltpu.make_async_copy(vals_hbm.at[pl.ds(wid * EW, CH)], vb, sv).wait()

    def s(j, t):
        vb, _ = sets[t]
        pltpu.sync_copy(vb, shared.at[idx_v.at[j]], add=True)

    v(0, 0)
    v(1, 1)

    def body(k, _):
        j = 2 * k
        wv(0)
        s(j, 0)
        v(j + 2, 0)
        wv(1)
        s(j + 1, 1)
        v(j + 3, 1)
        return _

    lax.fori_loop(0, (NCHUNK - 2) // 2, body, None)
    wv(0)
    s(NCHUNK - 2, 0)
    wv(1)
    s(NCHUNK - 1, 1)
    plsc.subcore_barrier()
    pltpu.sync_copy(shared.at[pl.ds(sid * NSPAN, NSPAN)], span)
    pltpu.sync_copy(span, out_hbm.at[cid, pl.ds(sid * NSPAN, NSPAN)])


@functools.partial(
    pl.kernel,
    out_type=jax.ShapeDtypeStruct((NC, N_PAD, FE), _f32),
    mesh=_mesh,
    scratch_types=(
        pltpu.VMEM((NCHUNK, CH), jnp.int32),
        pltpu.VMEM((CH, FE), _f32),
        pltpu.VMEM((NSPAN, FE), _f32),
        pltpu.VMEM_SHARED((N_PAD, FE), _f32),
    ),
    compiler_params=_sc_params,
)
def _sc_count(dst_hbm, out_hbm, idx_v, obuf, span, shared):
    cid = lax.axis_index("c")
    sid = lax.axis_index("s")
    wid = sid * NC + cid

    def zrow(i, _):
        span[i] = jnp.zeros((FE,), _f32)
        return _

    lax.fori_loop(0, NSPAN, zrow, None)

    def orow(i, _):
        obuf[i] = jnp.ones((FE,), _f32)
        return _

    lax.fori_loop(0, CH, orow, None)
    pltpu.sync_copy(span, shared.at[pl.ds(sid * NSPAN, NSPAN)])
    plsc.subcore_barrier()

    pltpu.sync_copy(dst_hbm.at[pl.ds(wid * NCHUNK, NCHUNK)], idx_v)

    def chunk(j, _):
        pltpu.sync_copy(obuf, shared.at[idx_v.at[j]], add=True)
        return _

    lax.fori_loop(0, NCHUNK, chunk, None)
    plsc.subcore_barrier()
    pltpu.sync_copy(shared.at[pl.ds(sid * NSPAN, NSPAN)], span)
    pltpu.sync_copy(span, out_hbm.at[cid, pl.ds(sid * NSPAN, NSPAN)])


NB = 1000
NGRID = N // NB
EP8 = E_PAD // 8
EB = 1280
EGRID = EP8 // EB


def _full(shape):
    return pl.BlockSpec(shape, lambda i: tuple(0 for _ in shape))


def _tc_edge_tables_body(x, b2d, uow, uot, a1, a2, au, gd, gs):
    oh = (b2d[...] == lax.broadcasted_iota(jnp.int32, (NB, NG), 1)).astype(_f32)
    tbl2 = _dotd(uot[...], a2[...])
    tblc = _dotd(uow[...], au[...])
    g = _dotd(x[...], a1[...]) + _dot(oh, tbl2)
    gd[...] = g
    gs[...] = _dot(oh, tblc) - g


_tc_edge_tables = pl.pallas_call(
    _tc_edge_tables_body,
    grid=(NGRID,),
    in_specs=[
        pl.BlockSpec((NB, FX), lambda i: (i, 0)),
        pl.BlockSpec((NB, 1), lambda i: (i, 0)),
        _full((NG, FU)), _full((NG, FU)),
        _full((FX, H)), _full((FU, H)), _full((FU, H)),
    ],
    out_specs=[pl.BlockSpec((NB, H), lambda i: (i, 0)),
               pl.BlockSpec((NB, H), lambda i: (i, 0))],
    out_shape=[jax.ShapeDtypeStruct((N, H), _f32),
               jax.ShapeDtypeStruct((N, H), _f32)],
)


def _tc_edge_mlp_body(rd, rs, ep, ae_bd, b1t, w2_bd, b2t, out):
    pre = rd[...] + rs[...] + _dotd(ep[...], ae_bd[...]) + b1t[...]
    out[...] = _dotd(jnp.maximum(pre, 0.0), w2_bd[...]) + b2t[...]


_tc_edge_mlp = pl.pallas_call(
    _tc_edge_mlp_body,
    grid=(EGRID,),
    in_specs=[
        pl.BlockSpec((EB, 8 * H), lambda i: (i, 0)),
        pl.BlockSpec((EB, 8 * H), lambda i: (i, 0)),
        pl.BlockSpec((EB, 8 * FE), lambda i: (i, 0)),
        _full((8 * FE, 8 * H)), _full((1, 8 * H)),
        _full((8 * H, 8 * FE)), _full((1, 8 * FE)),
    ],
    out_specs=pl.BlockSpec((EB, 8 * FE), lambda i: (i, 0)),
    out_shape=jax.ShapeDtypeStruct((EP8, 8 * FE), _f32),
)


def _tc_node_body(x, b2d, sp, cp, uow, uot,
                  b1w, b2w, buw, bew, b1n, w2n, b2n,
                  c1w, c2w, b1a, w2a1, w2a2, b2a1, b2a2,
                  cb, g1a, g1b, g1c, g2w, g2c,
                  xo, us1, us2, uo):
    i = pl.program_id(0)
    oh = (b2d[...] == lax.broadcasted_iota(jnp.int32, (NB, NG), 1)).astype(_f32)
    e_agg = (sp[0] + sp[1]) / jnp.clip(cp[0] + cp[1], 1.0, None)
    tbl_b = _dotd(uot[...], b2w[...]) + _dotd(uow[...], buw[...])
    x_pre = (_dotd(x[...], b1w[...]) + _dot(oh, tbl_b)
             + _dotd(e_agg, bew[...]) + b1n[...])
    x_new = _dotd(jnp.maximum(x_pre, 0.0), w2n[...]) + b2n[...]
    xo[...] = x_new
    ub = _dot(oh, uow[...])
    pre_a = _dotd(x_new, c1w[...]) + _dotd(ub, c2w[...]) + b1a[...]
    ah = jnp.maximum(pre_a, 0.0)
    a1 = jax.nn.sigmoid(_dotd(ah, w2a1[...]) + b2a1[...])
    a2 = jax.nn.sigmoid(_dotd(ah, w2a2[...]) + b2a2[...])
    w1 = a1 * x_new
    w2 = a2 * ub

    @pl.when(i == 0)
    def _init():
        us1[...] = jnp.zeros_like(us1)
        us2[...] = jnp.zeros_like(us2)

    us1[...] += _dott(oh, w1)
    us2[...] += _dott(oh, w2)

    @pl.when(i == NGRID - 1)
    def _glob():
        inv = 1.0 / jnp.clip(cb[...], 1.0, None)
        ua1 = us1[...] * inv
        ua2 = us2[...] * inv[:, :FU]
        pre = _dotd(ua1, g1a[...]) + _dotd(ua2, g1b[...]) + g1c[...]
        uo[...] = _dotd(jnp.maximum(pre, 0.0), g2w[...]) + g2c[...]


_tc_node = pl.pallas_call(
    _tc_node_body,
    grid=(NGRID,),
    in_specs=[
        pl.BlockSpec((NB, FX), lambda i: (i, 0)),
        pl.BlockSpec((NB, 1), lambda i: (i, 0)),
        pl.BlockSpec((NC, NB, FE), lambda i: (0, i, 0)),
        pl.BlockSpec((NC, NB, FE), lambda i: (0, i, 0)),
        _full((NG, FU)), _full((NG, FU)),
        _full((FX, H)), _full((FU, H)), _full((FU, H)), _full((FE, H)),
        _full((1, H)), _full((H, FX)), _full((1, FX)),
        _full((FX, H)), _full((FU, H)), _full((1, H)),
        _full((H, FX)), _full((H, FU)), _full((1, FX)), _full((1, FU)),
        _full((NG, NG)), _full((FX, H)), _full((FU, H)), _full((1, H)),
        _full((H, FU)), _full((1, FU)),
    ],
    out_specs=[pl.BlockSpec((NB, FX), lambda i: (i, 0)),
               _full((NG, FX)), _full((NG, FU)), _full((NG, FU))],
    out_shape=[jax.ShapeDtypeStruct((N, FX), _f32),
               jax.ShapeDtypeStruct((NG, FX), _f32),
               jax.ShapeDtypeStruct((NG, FU), _f32),
               jax.ShapeDtypeStruct((NG, FU), _f32)],
)


def _tc_final_body(u1, u2, f1a, f1b, f1c, f2w, f2c, out):
    pre = _dotd(u1[...], f1a[...]) + _dotd(u2[...], f1b[...]) + f1c[...]
    out[...] = _dotd(jnp.maximum(pre, 0.0), f2w[...]) + f2c[...]


_tc_final = pl.pallas_call(
    _tc_final_body,
    grid=(1,),
    in_specs=[_full((NG, FU)), _full((NG, FU)),
              _full((FU, H)), _full((FU, H)), _full((1, H)),
              _full((H, 64)), _full((1, 64))],
    out_specs=_full((NG, 64)),
    out_shape=jax.ShapeDtypeStruct((NG, 64), _f32),
)


def _tc_cntb_body(b2d, out):
    i = pl.program_id(0)
    oh = (b2d[...] == lax.broadcasted_iota(jnp.int32, (NB, NG), 1)).astype(_f32)

    @pl.when(i == 0)
    def _init():
        out[...] = jnp.zeros_like(out)

    out[...] += _dott(oh, jnp.ones((NB, NG), _f32))


_tc_cntb = pl.pallas_call(
    _tc_cntb_body,
    grid=(NGRID,),
    in_specs=[pl.BlockSpec((NB, 1), lambda i: (i, 0))],
    out_specs=_full((NG, NG)),
    out_shape=jax.ShapeDtypeStruct((NG, NG), _f32),
)


def _block_diag8(w):
    f_in, f_out = w.shape
    return (jnp.eye(8, dtype=_f32)[:, None, :, None] * w[None, :, None, :]
            ).reshape(8 * f_in, 8 * f_out)


def _pad_idx(idx, fill):
    pad = jnp.full((E_PAD - E,), fill, jnp.int32)
    return jnp.concatenate([idx, pad]).reshape(E_PAD // CH, CH)


def kernel(x1, edge_index1, e1, u1, batch1, x2, edge_index2, e2, u2, batch2,
           params):
    (w1e, b1e), (w2e, b2e) = params['edge']
    (w1n, b1n), (w2n, b2n) = params['node']
    (w1a, b1a), (w2a, b2a) = params['attn']
    (g1w, g1c), (g2w, g2c) = params['glob']
    (f1w, f1c), (f2w, f2c) = params['final']

    ae_bd = _block_diag8(w1e[160:176])
    w2e_bd = _block_diag8(w2e)
    b1e_t = jnp.tile(b1e, 8)[None, :]
    b2e_t = jnp.tile(b2e, 8)[None, :]

    edge_w = (w1e[0:128], w1e[128:160], w1e[176:208])
    node_w = (w1n[0:128], w1n[128:160], w1n[176:208], w1n[160:176],
              b1n[None, :], w2n, b2n[None, :])
    attn_w = (w1a[0:128], w1a[128:160], b1a[None, :],
              w2a[:, :128], w2a[:, 128:], b2a[None, :128], b2a[None, 128:])
    glob_w = (g1w[:128], g1w[128:], g1c[None, :], g2w, g2c[None, :])
    final_w = (f1w[:32], f1w[32:], f1c[None, :], f2w, f2c[None, :])

    def prep_graph(edge_index, batch, e):
        dst, src = edge_index[1], edge_index[0]
        dst_g = _pad_idx(dst, 0)
        src_g = _pad_idx(src, 0)
        dst_s = _pad_idx(dst, DUMP)
        b2d = batch[:, None]
        e_p = jnp.concatenate(
            [e, jnp.zeros((E_PAD - E, FE), _f32)]).reshape(EP8, 8 * FE)
        cnt = _sc_count(dst_s)
        cb = _tc_cntb(b2d)
        return dict(dst_g=dst_g, src_g=src_g, dst_s=dst_s, b2d=b2d,
                    e_p=e_p, cnt=cnt, cb=cb)

    g1 = prep_graph(edge_index1, batch1, e1)
    g2 = prep_graph(edge_index2, batch2, e2)

    def gnn(g, x, u_own, u_other):
        gd, gs = _tc_edge_tables(x, g['b2d'], u_own, u_other, *edge_w)
        rd, rs = _sc_gather(gd, gs, g['dst_g'], g['src_g'])
        e_new_p = _tc_edge_mlp(rd.reshape(EP8, 8 * H), rs.reshape(EP8, 8 * H),
                               g['e_p'], ae_bd, b1e_t, w2e_bd, b2e_t)
        s_part = _sc_scatter_add(e_new_p.reshape(E_PAD, FE), g['dst_s'])
        x_new, _, _, u_new = _tc_node(x, g['b2d'], s_part, g['cnt'],
                                      u_own, u_other, *node_w, *attn_w,
                                      g['cb'], *glob_w)
        return x_new, e_new_p, u_new

    outs = []
    for _ in range(2):
        x1, e_new1, u1 = gnn(g1, x1, u1, u2)
        g1['e_p'] = e_new1
        x2, e_new2, u2 = gnn(g2, x2, u2, u1)
        g2['e_p'] = e_new2
        outs.append(_tc_final(u1, u2, *final_w))
    return jnp.stack(outs)


# speedup vs baseline: 8.4785x; 1.0207x over previous
import functools

import jax
import jax.numpy as jnp
from jax import lax
from jax.experimental import pallas as pl
from jax.experimental.pallas import tpu as pltpu
from jax.experimental.pallas import tpu_sc as plsc

N = 10000
E = 320000
NG = 128
FE = 16
FX = 128
FU = 32
H = 32

NC, NS = 2, 16
NW = NC * NS
CH = 256
NCHUNK = 40
EW = CH * NCHUNK
E_PAD = EW * NW
N_PAD = 10240
DUMP = 10016
NSPAN = N_PAD // NS

_mesh = plsc.VectorSubcoreMesh(core_axis_name="c", subcore_axis_name="s",
                               num_cores=NC, num_subcores=NS)
_sc_params = pltpu.CompilerParams(use_tc_tiling_on_sc=False)
_f32 = jnp.float32
_HIGHEST = lax.Precision.HIGHEST


def _dot(a, b):
    return jax.lax.dot_general(a, b, (((a.ndim - 1,), (0,)), ((), ())),
                               precision=_HIGHEST, preferred_element_type=_f32)


def _dotd(a, b):
    return jax.lax.dot_general(a, b, (((a.ndim - 1,), (0,)), ((), ())),
                               preferred_element_type=_f32)


def _dott(a, b):
    return jax.lax.dot_general(a, b, (((0,), (0,)), ((), ())),
                               precision=_HIGHEST, preferred_element_type=_f32)


@functools.partial(
    pl.kernel,
    out_type=(jax.ShapeDtypeStruct((E_PAD, H), _f32),
              jax.ShapeDtypeStruct((E_PAD, H), _f32)),
    mesh=_mesh,
    scratch_types=(
        pltpu.VMEM((NCHUNK, CH), jnp.int32),
        pltpu.VMEM((NCHUNK, CH), jnp.int32),
        pltpu.VMEM((CH, H), _f32),
        pltpu.VMEM((CH, H), _f32),
        pltpu.VMEM((CH, H), _f32),
        pltpu.VMEM((CH, H), _f32),
        pltpu.SemaphoreType.DMA,
        pltpu.SemaphoreType.DMA,
        pltpu.SemaphoreType.DMA,
        pltpu.SemaphoreType.DMA,
        pltpu.SemaphoreType.DMA,
        pltpu.SemaphoreType.DMA,
        pltpu.SemaphoreType.DMA,
        pltpu.SemaphoreType.DMA,
    ),
    compiler_params=_sc_params,
)
def _sc_gather(gd_hbm, gs_hbm, dst_hbm, src_hbm, outd_hbm, outs_hbm,
               idx_d, idx_s, bd0, bs0, bd1, bs1,
               sgd0, sgs0, sgd1, sgs1, sod0, sos0, sod1, sos1):
    wid = lax.axis_index("s") * NC + lax.axis_index("c")
    base = wid * EW
    pltpu.sync_copy(dst_hbm.at[pl.ds(wid * NCHUNK, NCHUNK)], idx_d)
    pltpu.sync_copy(src_hbm.at[pl.ds(wid * NCHUNK, NCHUNK)], idx_s)

    sets = ((bd0, bs0, sgd0, sgs0, sod0, sos0),
            (bd1, bs1, sgd1, sgs1, sod1, sos1))

    def g(j, t):
        bd, bs, sgd, sgs, _, _ = sets[t]
        pltpu.async_copy(gd_hbm.at[idx_d.at[j]], bd, sgd)
        pltpu.async_copy(gs_hbm.at[idx_s.at[j]], bs, sgs)

    def wg(t):
        bd, bs, sgd, sgs, _, _ = sets[t]
        pltpu.make_async_copy(gd_hbm.at[pl.ds(0, CH)], bd, sgd).wait()
        pltpu.make_async_copy(gs_hbm.at[pl.ds(0, CH)], bs, sgs).wait()

    def o(j, t):
        bd, bs, _, _, sod, sos = sets[t]
        pltpu.async_copy(bd, outd_hbm.at[pl.ds(base + j * CH, CH)], sod)
        pltpu.async_copy(bs, outs_hbm.at[pl.ds(base + j * CH, CH)], sos)

    def wo(t):
        bd, bs, _, _, sod, sos = sets[t]
        pltpu.make_async_copy(bd, outd_hbm.at[pl.ds(base, CH)], sod).wait()
        pltpu.make_async_copy(bs, outs_hbm.at[pl.ds(base, CH)], sos).wait()

    g(0, 0)
    g(1, 1)
    wg(0)
    o(0, 0)

    def body(k, _):
        j = 2 * k + 1
        wo(0)
        g(j + 1, 0)
        wg(1)
        o(j, 1)
        wo(1)
        g(j + 2, 1)
        wg(0)
        o(j + 1, 0)
        return _

    lax.fori_loop(0, (NCHUNK - 2) // 2, body, None)
    wg(1)
    o(NCHUNK - 1, 1)
    wo(0)
    wo(1)


@functools.partial(
    pl.kernel,
    out_type=jax.ShapeDtypeStruct((NC, N_PAD, FE), _f32),
    mesh=_mesh,
    scratch_types=(
        pltpu.VMEM((NCHUNK, CH), jnp.int32),
        pltpu.VMEM((CH, FE), _f32),
        pltpu.VMEM((CH, FE), _f32),
        pltpu.VMEM((NSPAN, FE), _f32),
        pltpu.VMEM_SHARED((N_PAD, FE), _f32),
        pltpu.SemaphoreType.DMA,
        pltpu.SemaphoreType.DMA,
    ),
    compiler_params=_sc_params,
)
def _sc_scatter_add(vals_hbm, dst_hbm, out_hbm, idx_v, vb0, vb1, span, shared,
                    sv0, sv1):
    cid = lax.axis_index("c")
    sid = lax.axis_index("s")
    wid = sid * NC + cid

    def zrow(i, _):
        span[i] = jnp.zeros((FE,), _f32)
        return _

    lax.fori_loop(0, NSPAN, zrow, None)
    pltpu.sync_copy(span, shared.at[pl.ds(sid * NSPAN, NSPAN)])
    plsc.subcore_barrier()

    pltpu.sync_copy(dst_hbm.at[pl.ds(wid * NCHUNK, NCHUNK)], idx_v)

    sets = ((vb0, sv0), (vb1, sv1))

    def v(j, t):
        vb, sv = sets[t]
        pltpu.async_copy(vals_hbm.at[pl.ds(wid * EW + j * CH, CH)], vb, sv)

    def wv(t):
        vb, sv = sets[t]
        pltpu.make_async_copy(vals_hbm.at[pl.ds(wid * EW, CH)], vb, sv).wait()

    def s(j, t):
        vb, _ = sets[t]
        pltpu.sync_copy(vb, shared.at[idx_v.at[j]], add=True)

    v(0, 0)
    v(1, 1)

    def body(k, _):
        j = 2 * k
        wv(0)
        s(j, 0)
        v(j + 2, 0)
        wv(1)
        s(j + 1, 1)
        v(j + 3, 1)
        return _

    lax.fori_loop(0, (NCHUNK - 2) // 2, body, None)
    wv(0)
    s(NCHUNK - 2, 0)
    wv(1)
    s(NCHUNK - 1, 1)
    plsc.subcore_barrier()
    pltpu.sync_copy(shared.at[pl.ds(sid * NSPAN, NSPAN)], span)
    pltpu.sync_copy(span, out_hbm.at[cid, pl.ds(sid * NSPAN, NSPAN)])


@functools.partial(
    pl.kernel,
    out_type=jax.ShapeDtypeStruct((NC, N_PAD, FE), _f32),
    mesh=_mesh,
    scratch_types=(
        pltpu.VMEM((NCHUNK, CH), jnp.int32),
        pltpu.VMEM((CH, FE), _f32),
        pltpu.VMEM((NSPAN, FE), _f32),
        pltpu.VMEM_SHARED((N_PAD, FE), _f32),
    ),
    compiler_params=_sc_params,
)
def _sc_count(dst_hbm, out_hbm, idx_v, obuf, span, shared):
    cid = lax.axis_index("c")
    sid = lax.axis_index("s")
    wid = sid * NC + cid

    def zrow(i, _):
        span[i] = jnp.zeros((FE,), _f32)
        return _

    lax.fori_loop(0, NSPAN, zrow, None)

    def orow(i, _):
        obuf[i] = jnp.ones((FE,), _f32)
        return _

    lax.fori_loop(0, CH, orow, None)
    pltpu.sync_copy(span, shared.at[pl.ds(sid * NSPAN, NSPAN)])
    plsc.subcore_barrier()

    pltpu.sync_copy(dst_hbm.at[pl.ds(wid * NCHUNK, NCHUNK)], idx_v)

    def chunk(j, _):
        pltpu.sync_copy(obuf, shared.at[idx_v.at[j]], add=True)
        return _

    lax.fori_loop(0, NCHUNK, chunk, None)
    plsc.subcore_barrier()
    pltpu.sync_copy(shared.at[pl.ds(sid * NSPAN, NSPAN)], span)
    pltpu.sync_copy(span, out_hbm.at[cid, pl.ds(sid * NSPAN, NSPAN)])


NB = 1000
NGRID = N // NB
EP8 = E_PAD // 8
EB = 1280
EGRID = EP8 // EB


def _full(shape):
    return pl.BlockSpec(shape, lambda i: tuple(0 for _ in shape))


def _tc_edge_tables_body(x, b2d, uow, uot, a1, a2, au, gd, gs):
    oh = (b2d[...] == lax.broadcasted_iota(jnp.int32, (NB, NG), 1)).astype(_f32)
    tbl2 = _dotd(uot[...], a2[...])
    tblc = _dotd(uow[...], au[...])
    g = _dotd(x[...], a1[...]) + _dot(oh, tbl2)
    gd[...] = g
    gs[...] = _dot(oh, tblc) - g


_tc_edge_tables = pl.pallas_call(
    _tc_edge_tables_body,
    grid=(NGRID,),
    in_specs=[
        pl.BlockSpec((NB, FX), lambda i: (i, 0)),
        pl.BlockSpec((NB, 1), lambda i: (i, 0)),
        _full((NG, FU)), _full((NG, FU)),
        _full((FX, H)), _full((FU, H)), _full((FU, H)),
    ],
    out_specs=[pl.BlockSpec((NB, H), lambda i: (i, 0)),
               pl.BlockSpec((NB, H), lambda i: (i, 0))],
    out_shape=[jax.ShapeDtypeStruct((N, H), _f32),
               jax.ShapeDtypeStruct((N, H), _f32)],
)


def _tc_edge_mlp_body(rd, rs, ep, ae_bd, b1t, w2_bd, b2t, out):
    pre = rd[...] + rs[...] + _dotd(ep[...], ae_bd[...]) + b1t[...]
    out[...] = _dotd(jnp.maximum(pre, 0.0), w2_bd[...]) + b2t[...]


_tc_edge_mlp = pl.pallas_call(
    _tc_edge_mlp_body,
    grid=(EGRID,),
    in_specs=[
        pl.BlockSpec((EB, 8 * H), lambda i: (i, 0)),
        pl.BlockSpec((EB, 8 * H), lambda i: (i, 0)),
        pl.BlockSpec((EB, 8 * FE), lambda i: (i, 0)),
        _full((8 * FE, 8 * H)), _full((1, 8 * H)),
        _full((8 * H, 8 * FE)), _full((1, 8 * FE)),
    ],
    out_specs=pl.BlockSpec((EB, 8 * FE), lambda i: (i, 0)),
    out_shape=jax.ShapeDtypeStruct((EP8, 8 * FE), _f32),
)


def _tc_node_body(x, b2d, sp, cp, uow, uot,
                  b1w, b2w, buw, bew, b1n, w2n, b2n,
                  c1w, c2w, b1a, w2a1, w2a2, b2a1, b2a2,
                  cb, g1a, g1b, g1c, g2w, g2c,
                  xo, us1, us2, uo):
    i = pl.program_id(0)
    oh = (b2d[...] == lax.broadcasted_iota(jnp.int32, (NB, NG), 1)).astype(_f32)
    e_agg = (sp[0] + sp[1]) / jnp.clip(cp[0] + cp[1], 1.0, None)
    tbl_b = _dotd(uot[...], b2w[...]) + _dotd(uow[...], buw[...])
    x_pre = (_dotd(x[...], b1w[...]) + _dot(oh, tbl_b)
             + _dotd(e_agg, bew[...]) + b1n[...])
    x_new = _dotd(jnp.maximum(x_pre, 0.0), w2n[...]) + b2n[...]
    xo[...] = x_new
    ub = _dot(oh, uow[...])
    pre_a = _dotd(x_new, c1w[...]) + _dotd(ub, c2w[...]) + b1a[...]
    ah = jnp.maximum(pre_a, 0.0)
    a1 = jax.nn.sigmoid(_dotd(ah, w2a1[...]) + b2a1[...])
    a2 = jax.nn.sigmoid(_dotd(ah, w2a2[...]) + b2a2[...])
    w1 = a1 * x_new
    w2 = a2 * ub

    @pl.when(i == 0)
    def _init():
        us1[...] = jnp.zeros_like(us1)
        us2[...] = jnp.zeros_like(us2)

    us1[...] += _dott(oh, w1)
    us2[...] += _dott(oh, w2)

    @pl.when(i == NGRID - 1)
    def _glob():
        inv = 1.0 / jnp.clip(cb[...], 1.0, None)
        ua1 = us1[...] * inv
        ua2 = us2[...] * inv[:, :FU]
        pre = _dotd(ua1, g1a[...]) + _dotd(ua2, g1b[...]) + g1c[...]
        uo[...] = _dotd(jnp.maximum(pre, 0.0), g2w[...]) + g2c[...]


_tc_node = pl.pallas_call(
    _tc_node_body,
    grid=(NGRID,),
    in_specs=[
        pl.BlockSpec((NB, FX), lambda i: (i, 0)),
        pl.BlockSpec((NB, 1), lambda i: (i, 0)),
        pl.BlockSpec((NC, NB, FE), lambda i: (0, i, 0)),
        pl.BlockSpec((NC, NB, FE), lambda i: (0, i, 0)),
        _full((NG, FU)), _full((NG, FU)),
        _full((FX, H)), _full((FU, H)), _full((FU, H)), _full((FE, H)),
        _full((1, H)), _full((H, FX)), _full((1, FX)),
        _full((FX, H)), _full((FU, H)), _full((1, H)),
        _full((H, FX)), _full((H, FU)), _full((1, FX)), _full((1, FU)),
        _full((NG, NG)), _full((FX, H)), _full((FU, H)), _full((1, H)),
        _full((H, FU)), _full((1, FU)),
    ],
    out_specs=[pl.BlockSpec((NB, FX), lambda i: (i, 0)),
               _full((NG, FX)), _full((NG, FU)), _full((NG, FU))],
    out_shape=[jax.ShapeDtypeStruct((N, FX), _f32),
               jax.ShapeDtypeStruct((NG, FX), _f32),
               jax.ShapeDtypeStruct((NG, FU), _f32),
               jax.ShapeDtypeStruct((NG, FU), _f32)],
)


def _tc_final_body(u1, u2, f1a, f1b, f1c, f2w, f2c, out):
    pre = _dotd(u1[...], f1a[...]) + _dotd(u2[...], f1b[...]) + f1c[...]
    out[...] = _dotd(jnp.maximum(pre, 0.0), f2w[...]) + f2c[...]


_tc_final = pl.pallas_call(
    _tc_final_body,
    grid=(1,),
    in_specs=[_full((NG, FU)), _full((NG, FU)),
              _full((FU, H)), _full((FU, H)), _full((1, H)),
              _full((H, 64)), _full((1, 64))],
    out_specs=_full((NG, 64)),
    out_shape=jax.ShapeDtypeStruct((NG, 64), _f32),
)


def _tc_cntb_body(b2d, out):
    i = pl.program_id(0)
    oh = (b2d[...] == lax.broadcasted_iota(jnp.int32, (NB, NG), 1)).astype(_f32)

    @pl.when(i == 0)
    def _init():
        out[...] = jnp.zeros_like(out)

    out[...] += _dott(oh, jnp.ones((NB, NG), _f32))


_tc_cntb = pl.pallas_call(
    _tc_cntb_body,
    grid=(NGRID,),
    in_specs=[pl.BlockSpec((NB, 1), lambda i: (i, 0))],
    out_specs=_full((NG, NG)),
    out_shape=jax.ShapeDtypeStruct((NG, NG), _f32),
)


def _block_diag8(w):
    f_in, f_out = w.shape
    return (jnp.eye(8, dtype=_f32)[:, None, :, None] * w[None, :, None, :]
            ).reshape(8 * f_in, 8 * f_out)


def _pad_idx(idx, fill):
    pad = jnp.full((E_PAD - E,), fill, jnp.int32)
    return jnp.concatenate([idx, pad]).reshape(E_PAD // CH, CH)


def kernel(x1, edge_index1, e1, u1, batch1, x2, edge_index2, e2, u2, batch2,
           params):
    (w1e, b1e), (w2e, b2e) = params['edge']
    (w1n, b1n), (w2n, b2n) = params['node']
    (w1a, b1a), (w2a, b2a) = params['attn']
    (g1w, g1c), (g2w, g2c) = params['glob']
    (f1w, f1c), (f2w, f2c) = params['final']

    ae_bd = _block_diag8(w1e[160:176])
    w2e_bd = _block_diag8(w2e)
    b1e_t = jnp.tile(b1e, 8)[None, :]
    b2e_t = jnp.tile(b2e, 8)[None, :]

    edge_w = (w1e[0:128], w1e[128:160], w1e[176:208])
    node_w = (w1n[0:128], w1n[128:160], w1n[176:208], w1n[160:176],
              b1n[None, :], w2n, b2n[None, :])
    attn_w = (w1a[0:128], w1a[128:160], b1a[None, :],
              w2a[:, :128], w2a[:, 128:], b2a[None, :128], b2a[None, 128:])
    glob_w = (g1w[:128], g1w[128:], g1c[None, :], g2w, g2c[None, :])
    final_w = (f1w[:32], f1w[32:], f1c[None, :], f2w, f2c[None, :])

    def prep_graph(edge_index, batch, e):
        dst, src = edge_index[1], edge_index[0]
        dst_g = _pad_idx(dst, 0)
        src_g = _pad_idx(src, 0)
        dst_s = _pad_idx(dst, DUMP)
        b2d = batch[:, None]
        e_p = jnp.concatenate(
            [e, jnp.zeros((E_PAD - E, FE), _f32)]).reshape(EP8, 8 * FE)
        cnt = _sc_count(dst_s)
        cb = _tc_cntb(b2d)
        return dict(dst_g=dst_g, src_g=src_g, dst_s=dst_s, b2d=b2d,
                    e_p=e_p, cnt=cnt, cb=cb)

    g1 = prep_graph(edge_index1, batch1, e1)
    g2 = prep_graph(edge_index2, batch2, e2)

    def gnn(g, x, u_own, u_other):
        gd, gs = _tc_edge_tables(x, g['b2d'], u_own, u_other, *edge_w)
        rd, rs = _sc_gather(gd, gs, g['dst_g'], g['src_g'])
        e_new_p = _tc_edge_mlp(rd.reshape(EP8, 8 * H), rs.reshape(EP8, 8 * H),
                               g['e_p'], ae_bd, b1e_t, w2e_bd, b2e_t)
        s_part = _sc_scatter_add(e_new_p.reshape(E_PAD, FE), g['dst_s'])
        x_new, _, _, u_new = _tc_node(x, g['b2d'], s_part, g['cnt'],
                                      u_own, u_other, *node_w, *attn_w,
                                      g['cb'], *glob_w)
        return x_new, e_new_p, u_new

    outs = []
    for _ in range(2):
        x1, e_new1, u1 = gnn(g1, x1, u1, u2)
        g1['e_p'] = e_new1
        x2, e_new2, u2 = gnn(g2, x2, u2, u1)
        g2['e_p'] = e_new2
        outs.append(_tc_final(u1, u2, *final_w))
    return jnp.stack(outs)


# speedup vs baseline: 8.5051x; 1.0031x over previous
import functools

import jax
import jax.numpy as jnp
from jax import lax
from jax.experimental import pallas as pl
from jax.experimental.pallas import tpu as pltpu
from jax.experimental.pallas import tpu_sc as plsc

N = 10000
E = 320000
NG = 128
FE = 16
FX = 128
FU = 32
H = 32

NC, NS = 2, 16
NW = NC * NS
CH = 320
NCHUNK = 32
EW = CH * NCHUNK
E_PAD = EW * NW
N_PAD = 10240
DUMP = 10016
NSPAN = N_PAD // NS

_mesh = plsc.VectorSubcoreMesh(core_axis_name="c", subcore_axis_name="s",
                               num_cores=NC, num_subcores=NS)
_sc_params = pltpu.CompilerParams(use_tc_tiling_on_sc=False)
_f32 = jnp.float32
_HIGHEST = lax.Precision.HIGHEST


def _dot(a, b):
    return jax.lax.dot_general(a, b, (((a.ndim - 1,), (0,)), ((), ())),
                               precision=_HIGHEST, preferred_element_type=_f32)


def _dotd(a, b):
    return jax.lax.dot_general(a, b, (((a.ndim - 1,), (0,)), ((), ())),
                               preferred_element_type=_f32)


def _dott(a, b):
    return jax.lax.dot_general(a, b, (((0,), (0,)), ((), ())),
                               precision=_HIGHEST, preferred_element_type=_f32)


@functools.partial(
    pl.kernel,
    out_type=(jax.ShapeDtypeStruct((E_PAD, H), _f32),
              jax.ShapeDtypeStruct((E_PAD, H), _f32)),
    mesh=_mesh,
    scratch_types=(
        pltpu.VMEM((NCHUNK, CH), jnp.int32),
        pltpu.VMEM((NCHUNK, CH), jnp.int32),
        pltpu.VMEM((CH, H), _f32),
        pltpu.VMEM((CH, H), _f32),
        pltpu.VMEM((CH, H), _f32),
        pltpu.VMEM((CH, H), _f32),
        pltpu.SemaphoreType.DMA,
        pltpu.SemaphoreType.DMA,
        pltpu.SemaphoreType.DMA,
        pltpu.SemaphoreType.DMA,
        pltpu.SemaphoreType.DMA,
        pltpu.SemaphoreType.DMA,
        pltpu.SemaphoreType.DMA,
        pltpu.SemaphoreType.DMA,
    ),
    compiler_params=_sc_params,
)
def _sc_gather(gd_hbm, gs_hbm, dst_hbm, src_hbm, outd_hbm, outs_hbm,
               idx_d, idx_s, bd0, bs0, bd1, bs1,
               sgd0, sgs0, sgd1, sgs1, sod0, sos0, sod1, sos1):
    wid = lax.axis_index("s") * NC + lax.axis_index("c")
    base = wid * EW
    pltpu.sync_copy(dst_hbm.at[pl.ds(wid * NCHUNK, NCHUNK)], idx_d)
    pltpu.sync_copy(src_hbm.at[pl.ds(wid * NCHUNK, NCHUNK)], idx_s)

    sets = ((bd0, bs0, sgd0, sgs0, sod0, sos0),
            (bd1, bs1, sgd1, sgs1, sod1, sos1))

    def g(j, t):
        bd, bs, sgd, sgs, _, _ = sets[t]
        pltpu.async_copy(gd_hbm.at[idx_d.at[j]], bd, sgd)
        pltpu.async_copy(gs_hbm.at[idx_s.at[j]], bs, sgs)

    def wg(t):
        bd, bs, sgd, sgs, _, _ = sets[t]
        pltpu.make_async_copy(gd_hbm.at[pl.ds(0, CH)], bd, sgd).wait()
        pltpu.make_async_copy(gs_hbm.at[pl.ds(0, CH)], bs, sgs).wait()

    def o(j, t):
        bd, bs, _, _, sod, sos = sets[t]
        pltpu.async_copy(bd, outd_hbm.at[pl.ds(base + j * CH, CH)], sod)
        pltpu.async_copy(bs, outs_hbm.at[pl.ds(base + j * CH, CH)], sos)

    def wo(t):
        bd, bs, _, _, sod, sos = sets[t]
        pltpu.make_async_copy(bd, outd_hbm.at[pl.ds(base, CH)], sod).wait()
        pltpu.make_async_copy(bs, outs_hbm.at[pl.ds(base, CH)], sos).wait()

    g(0, 0)
    g(1, 1)
    wg(0)
    o(0, 0)

    def body(k, _):
        j = 2 * k + 1
        wo(0)
        g(j + 1, 0)
        wg(1)
        o(j, 1)
        wo(1)
        g(j + 2, 1)
        wg(0)
        o(j + 1, 0)
        return _

    lax.fori_loop(0, (NCHUNK - 2) // 2, body, None)
    wg(1)
    o(NCHUNK - 1, 1)
    wo(0)
    wo(1)


@functools.partial(
    pl.kernel,
    out_type=jax.ShapeDtypeStruct((NC, N_PAD, FE), _f32),
    mesh=_mesh,
    scratch_types=(
        pltpu.VMEM((NCHUNK, CH), jnp.int32),
        pltpu.VMEM((CH, FE), _f32),
        pltpu.VMEM((CH, FE), _f32),
        pltpu.VMEM((NSPAN, FE), _f32),
        pltpu.VMEM_SHARED((N_PAD, FE), _f32),
        pltpu.SemaphoreType.DMA,
        pltpu.SemaphoreType.DMA,
    ),
    compiler_params=_sc_params,
)
def _sc_scatter_add(vals_hbm, dst_hbm, out_hbm, idx_v, vb0, vb1, span, shared,
                    sv0, sv1):
    cid = lax.axis_index("c")
    sid = lax.axis_index("s")
    wid = sid * NC + cid

    def zrow(i, _):
        span[i] = jnp.zeros((FE,), _f32)
        return _

    lax.fori_loop(0, NSPAN, zrow, None)
    pltpu.sync_copy(span, shared.at[pl.ds(sid * NSPAN, NSPAN)])
    plsc.subcore_barrier()

    pltpu.sync_copy(dst_hbm.at[pl.ds(wid * NCHUNK, NCHUNK)], idx_v)

    sets = ((vb0, sv0), (vb1, sv1))

    def v(j, t):
        vb, sv = sets[t]
        pltpu.async_copy(vals_hbm.at[pl.ds(wid * EW + j * CH, CH)], vb, sv)

    def wv(t):
        vb, sv = sets[t]
        pltpu.make_async_copy(vals_hbm.at[pl.ds(wid * EW, CH)], vb, sv).wait()

    def s(j, t):
        vb, _ = sets[t]
        pltpu.sync_copy(vb, shared.at[idx_v.at[j]], add=True)

    v(0, 0)
    v(1, 1)

    def body(k, _):
        j = 2 * k
        wv(0)
        s(j, 0)
        v(j + 2, 0)
        wv(1)
        s(j + 1, 1)
        v(j + 3, 1)
        return _

    lax.fori_loop(0, (NCHUNK - 2) // 2, body, None)
    wv(0)
    s(NCHUNK - 2, 0)
    wv(1)
    s(NCHUNK - 1, 1)
    plsc.subcore_barrier()
    pltpu.sync_copy(shared.at[pl.ds(sid * NSPAN, NSPAN)], span)
    pltpu.sync_copy(span, out_hbm.at[cid, pl.ds(sid * NSPAN, NSPAN)])


@functools.partial(
    pl.kernel,
    out_type=jax.ShapeDtypeStruct((NC, N_PAD, FE), _f32),
    mesh=_mesh,
    scratch_types=(
        pltpu.VMEM((NCHUNK, CH), jnp.int32),
        pltpu.VMEM((CH, FE), _f32),
        pltpu.VMEM((NSPAN, FE), _f32),
        pltpu.VMEM_SHARED((N_PAD, FE), _f32),
    ),
    compiler_params=_sc_params,
)
def _sc_count(dst_hbm, out_hbm, idx_v, obuf, span, shared):
    cid = lax.axis_index("c")
    sid = lax.axis_index("s")
    wid = sid * NC + cid

    def zrow(i, _):
        span[i] = jnp.zeros((FE,), _f32)
        return _

    lax.fori_loop(0, NSPAN, zrow, None)

    def orow(i, _):
        obuf[i] = jnp.ones((FE,), _f32)
        return _

    lax.fori_loop(0, CH, orow, None)
    pltpu.sync_copy(span, shared.at[pl.ds(sid * NSPAN, NSPAN)])
    plsc.subcore_barrier()

    pltpu.sync_copy(dst_hbm.at[pl.ds(wid * NCHUNK, NCHUNK)], idx_v)

    def chunk(j, _):
        pltpu.sync_copy(obuf, shared.at[idx_v.at[j]], add=True)
        return _

    lax.fori_loop(0, NCHUNK, chunk, None)
    plsc.subcore_barrier()
    pltpu.sync_copy(shared.at[pl.ds(sid * NSPAN, NSPAN)], span)
    pltpu.sync_copy(span, out_hbm.at[cid, pl.ds(sid * NSPAN, NSPAN)])


NB = 1000
NGRID = N // NB
EP8 = E_PAD // 8
EB = 1280
EGRID = EP8 // EB


def _full(shape):
    return pl.BlockSpec(shape, lambda i: tuple(0 for _ in shape))


def _tc_edge_tables_body(x, b2d, uow, uot, a1, a2, au, gd, gs):
    oh = (b2d[...] == lax.broadcasted_iota(jnp.int32, (NB, NG), 1)).astype(_f32)
    tbl2 = _dotd(uot[...], a2[...])
    tblc = _dotd(uow[...], au[...])
    g = _dotd(x[...], a1[...]) + _dot(oh, tbl2)
    gd[...] = g
    gs[...] = _dot(oh, tblc) - g


_tc_edge_tables = pl.pallas_call(
    _tc_edge_tables_body,
    grid=(NGRID,),
    in_specs=[
        pl.BlockSpec((NB, FX), lambda i: (i, 0)),
        pl.BlockSpec((NB, 1), lambda i: (i, 0)),
        _full((NG, FU)), _full((NG, FU)),
        _full((FX, H)), _full((FU, H)), _full((FU, H)),
    ],
    out_specs=[pl.BlockSpec((NB, H), lambda i: (i, 0)),
               pl.BlockSpec((NB, H), lambda i: (i, 0))],
    out_shape=[jax.ShapeDtypeStruct((N, H), _f32),
               jax.ShapeDtypeStruct((N, H), _f32)],
)


def _tc_edge_mlp_body(rd, rs, ep, ae_bd, b1t, w2_bd, b2t, out):
    pre = rd[...] + rs[...] + _dotd(ep[...], ae_bd[...]) + b1t[...]
    out[...] = _dotd(jnp.maximum(pre, 0.0), w2_bd[...]) + b2t[...]


_tc_edge_mlp = pl.pallas_call(
    _tc_edge_mlp_body,
    grid=(EGRID,),
    in_specs=[
        pl.BlockSpec((EB, 8 * H), lambda i: (i, 0)),
        pl.BlockSpec((EB, 8 * H), lambda i: (i, 0)),
        pl.BlockSpec((EB, 8 * FE), lambda i: (i, 0)),
        _full((8 * FE, 8 * H)), _full((1, 8 * H)),
        _full((8 * H, 8 * FE)), _full((1, 8 * FE)),
    ],
    out_specs=pl.BlockSpec((EB, 8 * FE), lambda i: (i, 0)),
    out_shape=jax.ShapeDtypeStruct((EP8, 8 * FE), _f32),
)


def _tc_node_body(x, b2d, sp, cp, uow, uot,
                  b1w, b2w, buw, bew, b1n, w2n, b2n,
                  c1w, c2w, b1a, w2a1, w2a2, b2a1, b2a2,
                  cb, g1a, g1b, g1c, g2w, g2c,
                  xo, us1, us2, uo):
    i = pl.program_id(0)
    oh = (b2d[...] == lax.broadcasted_iota(jnp.int32, (NB, NG), 1)).astype(_f32)
    e_agg = (sp[0] + sp[1]) / jnp.clip(cp[0] + cp[1], 1.0, None)
    tbl_b = _dotd(uot[...], b2w[...]) + _dotd(uow[...], buw[...])
    x_pre = (_dotd(x[...], b1w[...]) + _dot(oh, tbl_b)
             + _dotd(e_agg, bew[...]) + b1n[...])
    x_new = _dotd(jnp.maximum(x_pre, 0.0), w2n[...]) + b2n[...]
    xo[...] = x_new
    ub = _dot(oh, uow[...])
    pre_a = _dotd(x_new, c1w[...]) + _dotd(ub, c2w[...]) + b1a[...]
    ah = jnp.maximum(pre_a, 0.0)
    a1 = jax.nn.sigmoid(_dotd(ah, w2a1[...]) + b2a1[...])
    a2 = jax.nn.sigmoid(_dotd(ah, w2a2[...]) + b2a2[...])
    w1 = a1 * x_new
    w2 = a2 * ub

    @pl.when(i == 0)
    def _init():
        us1[...] = jnp.zeros_like(us1)
        us2[...] = jnp.zeros_like(us2)

    us1[...] += _dott(oh, w1)
    us2[...] += _dott(oh, w2)

    @pl.when(i == NGRID - 1)
    def _glob():
        inv = 1.0 / jnp.clip(cb[...], 1.0, None)
        ua1 = us1[...] * inv
        ua2 = us2[...] * inv[:, :FU]
        pre = _dotd(ua1, g1a[...]) + _dotd(ua2, g1b[...]) + g1c[...]
        uo[...] = _dotd(jnp.maximum(pre, 0.0), g2w[...]) + g2c[...]


_tc_node = pl.pallas_call(
    _tc_node_body,
    grid=(NGRID,),
    in_specs=[
        pl.BlockSpec((NB, FX), lambda i: (i, 0)),
        pl.BlockSpec((NB, 1), lambda i: (i, 0)),
        pl.BlockSpec((NC, NB, FE), lambda i: (0, i, 0)),
        pl.BlockSpec((NC, NB, FE), lambda i: (0, i, 0)),
        _full((NG, FU)), _full((NG, FU)),
        _full((FX, H)), _full((FU, H)), _full((FU, H)), _full((FE, H)),
        _full((1, H)), _full((H, FX)), _full((1, FX)),
        _full((FX, H)), _full((FU, H)), _full((1, H)),
        _full((H, FX)), _full((H, FU)), _full((1, FX)), _full((1, FU)),
        _full((NG, NG)), _full((FX, H)), _full((FU, H)), _full((1, H)),
        _full((H, FU)), _full((1, FU)),
    ],
    out_specs=[pl.BlockSpec((NB, FX), lambda i: (i, 0)),
               _full((NG, FX)), _full((NG, FU)), _full((NG, FU))],
    out_shape=[jax.ShapeDtypeStruct((N, FX), _f32),
               jax.ShapeDtypeStruct((NG, FX), _f32),
               jax.ShapeDtypeStruct((NG, FU), _f32),
               jax.ShapeDtypeStruct((NG, FU), _f32)],
)


def _tc_final_body(u1, u2, f1a, f1b, f1c, f2w, f2c, out):
    pre = _dotd(u1[...], f1a[...]) + _dotd(u2[...], f1b[...]) + f1c[...]
    out[...] = _dotd(jnp.maximum(pre, 0.0), f2w[...]) + f2c[...]


_tc_final = pl.pallas_call(
    _tc_final_body,
    grid=(1,),
    in_specs=[_full((NG, FU)), _full((NG, FU)),
              _full((FU, H)), _full((FU, H)), _full((1, H)),
              _full((H, 64)), _full((1, 64))],
    out_specs=_full((NG, 64)),
    out_shape=jax.ShapeDtypeStruct((NG, 64), _f32),
)


def _tc_cntb_body(b2d, out):
    i = pl.program_id(0)
    oh = (b2d[...] == lax.broadcasted_iota(jnp.int32, (NB, NG), 1)).astype(_f32)

    @pl.when(i == 0)
    def _init():
        out[...] = jnp.zeros_like(out)

    out[...] += _dott(oh, jnp.ones((NB, NG), _f32))


_tc_cntb = pl.pallas_call(
    _tc_cntb_body,
    grid=(NGRID,),
    in_specs=[pl.BlockSpec((NB, 1), lambda i: (i, 0))],
    out_specs=_full((NG, NG)),
    out_shape=jax.ShapeDtypeStruct((NG, NG), _f32),
)


def _block_diag8(w):
    f_in, f_out = w.shape
    return (jnp.eye(8, dtype=_f32)[:, None, :, None] * w[None, :, None, :]
            ).reshape(8 * f_in, 8 * f_out)


def _pad_idx(idx, fill):
    pad = jnp.full((E_PAD - E,), fill, jnp.int32)
    return jnp.concatenate([idx, pad]).reshape(E_PAD // CH, CH)


def kernel(x1, edge_index1, e1, u1, batch1, x2, edge_index2, e2, u2, batch2,
           params):
    (w1e, b1e), (w2e, b2e) = params['edge']
    (w1n, b1n), (w2n, b2n) = params['node']
    (w1a, b1a), (w2a, b2a) = params['attn']
    (g1w, g1c), (g2w, g2c) = params['glob']
    (f1w, f1c), (f2w, f2c) = params['final']

    ae_bd = _block_diag8(w1e[160:176])
    w2e_bd = _block_diag8(w2e)
    b1e_t = jnp.tile(b1e, 8)[None, :]
    b2e_t = jnp.tile(b2e, 8)[None, :]

    edge_w = (w1e[0:128], w1e[128:160], w1e[176:208])
    node_w = (w1n[0:128], w1n[128:160], w1n[176:208], w1n[160:176],
              b1n[None, :], w2n, b2n[None, :])
    attn_w = (w1a[0:128], w1a[128:160], b1a[None, :],
              w2a[:, :128], w2a[:, 128:], b2a[None, :128], b2a[None, 128:])
    glob_w = (g1w[:128], g1w[128:], g1c[None, :], g2w, g2c[None, :])
    final_w = (f1w[:32], f1w[32:], f1c[None, :], f2w, f2c[None, :])

    def prep_graph(edge_index, batch, e):
        dst, src = edge_index[1], edge_index[0]
        dst_g = _pad_idx(dst, 0)
        src_g = _pad_idx(src, 0)
        dst_s = _pad_idx(dst, DUMP)
        b2d = batch[:, None]
        e_p = jnp.concatenate(
            [e, jnp.zeros((E_PAD - E, FE), _f32)]).reshape(EP8, 8 * FE)
        cnt = _sc_count(dst_s)
        cb = _tc_cntb(b2d)
        return dict(dst_g=dst_g, src_g=src_g, dst_s=dst_s, b2d=b2d,
                    e_p=e_p, cnt=cnt, cb=cb)

    g1 = prep_graph(edge_index1, batch1, e1)
    g2 = prep_graph(edge_index2, batch2, e2)

    def gnn(g, x, u_own, u_other):
        gd, gs = _tc_edge_tables(x, g['b2d'], u_own, u_other, *edge_w)
        rd, rs = _sc_gather(gd, gs, g['dst_g'], g['src_g'])
        e_new_p = _tc_edge_mlp(rd.reshape(EP8, 8 * H), rs.reshape(EP8, 8 * H),
                               g['e_p'], ae_bd, b1e_t, w2e_bd, b2e_t)
        s_part = _sc_scatter_add(e_new_p.reshape(E_PAD, FE), g['dst_s'])
        x_new, _, _, u_new = _tc_node(x, g['b2d'], s_part, g['cnt'],
                                      u_own, u_other, *node_w, *attn_w,
                                      g['cb'], *glob_w)
        return x_new, e_new_p, u_new

    outs = []
    for _ in range(2):
        x1, e_new1, u1 = gnn(g1, x1, u1, u2)
        g1['e_p'] = e_new1
        x2, e_new2, u2 = gnn(g2, x2, u2, u1)
        g2['e_p'] = e_new2
        outs.append(_tc_final(u1, u2, *final_w))
    return jnp.stack(outs)
